# baseline (speedup 1.0000x reference)
import jax
import jax.numpy as jnp
from jax.experimental import pallas as pl
from jax.experimental.pallas import tpu as pltpu

_BM = 128


def _mm(a_ref, v_ref, o_ref):
    o_ref[...] = jnp.dot(a_ref[...], v_ref[...], preferred_element_type=jnp.float32)


def kernel(matrix, matrix_batch):
    m, k = matrix.shape
    b, _, f = matrix_batch.shape
    n = b * f
    vectors = jnp.swapaxes(matrix_batch, 0, 1).reshape(k, n)

    out = pl.pallas_call(
        _mm,
        grid=(m // _BM,),
        in_specs=[
            pl.BlockSpec((_BM, k), lambda i: (i, 0)),
            pl.BlockSpec((k, n), lambda i: (0, 0)),
        ],
        out_specs=pl.BlockSpec((_BM, n), lambda i: (i, 0)),
        out_shape=jax.ShapeDtypeStruct((m, n), jnp.float32),
    )(matrix, vectors)

    return jnp.swapaxes(out.reshape(m, b, f), 0, 1)


# speedup vs baseline: 1.1975x; 1.1975x over previous
import jax
import jax.numpy as jnp
from jax.experimental import pallas as pl
from jax.experimental.pallas import tpu as pltpu

_BM = 256


def _mm(a_ref, v_ref, o_ref):
    o_ref[...] = jnp.dot(a_ref[...], v_ref[...], preferred_element_type=jnp.float32)


def kernel(matrix, matrix_batch):
    m, k = matrix.shape
    b, _, f = matrix_batch.shape
    n = b * f
    vectors = jnp.swapaxes(matrix_batch, 0, 1).reshape(k, n)

    out = pl.pallas_call(
        _mm,
        grid=(m // _BM,),
        in_specs=[
            pl.BlockSpec((_BM, k), lambda i: (i, 0)),
            pl.BlockSpec((k, n), lambda i: (0, 0)),
        ],
        out_specs=pl.BlockSpec((_BM, n), lambda i: (i, 0)),
        out_shape=jax.ShapeDtypeStruct((m, n), jnp.float32),
    )(matrix, vectors)

    return jnp.swapaxes(out.reshape(m, b, f), 0, 1)
